# baseline (speedup 1.0000x reference)
import jax
import jax.numpy as jnp
from jax import lax
from jax.experimental import pallas as pl
from jax.experimental.pallas import tpu as pltpu
from jax.experimental.pallas import tpu_sc as plsc

_S = 64.0
_M = 0.4

_NC = 2
_NS = 16
_NW = _NC * _NS
_CH = 1536
_NBUF = 4
_C_SC = 98304
_TAIL_BLOCK = 2048
_UNROLL = 16


def _sc_scale(logits, labx):
    b, c = logits.shape
    rows_per_w = b // _NW
    nch = _C_SC // _CH
    ngrp = rows_per_w // 8
    t_total = ngrp * nch

    def body(logits_ref, labx_ref, out_ref, labx_v, *scr):
        cid = lax.axis_index("c")
        sid = lax.axis_index("s")
        wid = sid * _NC + cid
        r0 = wid * rows_per_w
        pltpu.sync_copy(
            labx_ref.at[pl.ds(pl.multiple_of(r0, 8), rows_per_w)], labx_v)

        ins = scr[0:_NBUF]
        outs = scr[_NBUF:2 * _NBUF]
        lsems = scr[2 * _NBUF:3 * _NBUF]
        ssems = scr[3 * _NBUF:4 * _NBUF]

        def coords(t):
            rg = t // nch
            row8 = pl.multiple_of(r0 + rg * 8, 8)
            c0 = pl.multiple_of((t % nch) * _CH, 128)
            return rg, row8, c0

        def src_slice(t):
            _, row8, c0 = coords(t)
            return logits_ref.at[pl.ds(row8, 8), pl.ds(c0, _CH)]

        def dst_slice(t):
            _, row8, c0 = coords(t)
            return out_ref.at[pl.ds(row8, 8), pl.ds(c0, _CH)]

        for bb in range(_NBUF):
            pltpu.async_copy(src_slice(bb), ins[bb], lsems[bb])

        lanes = lax.broadcasted_iota(jnp.int32, (16,), 0)

        def group(g, carry):
            for bb in range(_NBUF):
                t = g * _NBUF + bb
                ib, ob, ls, ss = ins[bb], outs[bb], lsems[bb], ssems[bb]
                pltpu.make_async_copy(src_slice(t), ib, ls).wait()

                @pl.when(t >= _NBUF)
                def _(ob=ob, ss=ss, t=t):
                    pltpu.make_async_copy(ob, dst_slice(t - _NBUF), ss).wait()

                rg, _, c0 = coords(t)

                def row_fn(r, carry2, ib=ib, ob=ob, c0=c0, rg=rg):
                    labsplat = labx_v[rg * 8 + r, pl.ds(0, 16)]
                    lrel = labsplat - c0

                    def scale(j, carry3):
                        base = j * (16 * _UNROLL)
                        vs = [ib[r, pl.ds(base + u * 16, 16)]
                              for u in range(_UNROLL)]
                        for u in range(_UNROLL):
                            m = (lrel - (base + u * 16)) == lanes
                            ob[r, pl.ds(base + u * 16, 16)] = (
                                vs[u] - jnp.where(m, _M, 0.0)) * _S
                        return carry3

                    lax.fori_loop(0, _CH // (16 * _UNROLL), scale, 0)
                    return carry2

                lax.fori_loop(0, 8, row_fn, 0)

                pltpu.async_copy(ob, dst_slice(t), ss)

                @pl.when(t + _NBUF < t_total)
                def _(ib=ib, ls=ls, t=t):
                    pltpu.async_copy(src_slice(t + _NBUF), ib, ls)

            return carry

        lax.fori_loop(0, t_total // _NBUF, group, 0)
        for bb in range(_NBUF):
            pltpu.make_async_copy(
                outs[bb], dst_slice(t_total - _NBUF + bb), ssems[bb]).wait()

    mesh = plsc.VectorSubcoreMesh(
        core_axis_name="c", subcore_axis_name="s",
        num_cores=_NC, num_subcores=_NS,
    )
    fn = pl.kernel(
        body,
        out_type=jax.ShapeDtypeStruct((b, c), jnp.float32),
        mesh=mesh,
        scratch_types=(
            [pltpu.VMEM((rows_per_w, 128), jnp.int32)]
            + [pltpu.VMEM((8, _CH), jnp.float32)] * (2 * _NBUF)
            + [pltpu.SemaphoreType.DMA] * (2 * _NBUF)
        ),
    )
    return fn(logits, labx)


def _tail_body(alias_ref, labels_ref, x_ref, o_ref):
    del alias_ref
    bb, bc = x_ref.shape
    cols = _C_SC + jax.lax.broadcasted_iota(jnp.int32, (bb, bc), 1)
    lab = labels_ref[...]
    x = x_ref[...]
    o_ref[...] = (x - jnp.where(cols == lab, _M, 0.0)) * _S


def _tc_tail(sc_out, logits, labels_i32):
    b, c = logits.shape
    jblk = _C_SC // _TAIL_BLOCK
    block_b = 16
    labels2d = labels_i32.reshape(b, 1)
    return pl.pallas_call(
        _tail_body,
        grid=(b // block_b,),
        in_specs=[
            pl.BlockSpec(memory_space=pltpu.HBM),
            pl.BlockSpec((block_b, 1), lambda i: (i, 0)),
            pl.BlockSpec((block_b, _TAIL_BLOCK), lambda i: (i, jblk)),
        ],
        out_specs=pl.BlockSpec((block_b, _TAIL_BLOCK), lambda i: (i, jblk)),
        out_shape=jax.ShapeDtypeStruct((b, c), jnp.float32),
        input_output_aliases={0: 0},
    )(sc_out, labels2d, logits)


def kernel(logits, norms, labels):
    del norms
    b, _ = logits.shape
    labels_i32 = labels.astype(jnp.int32)
    labx = jnp.broadcast_to(labels_i32.reshape(b, 1), (b, 128))
    sc_out = _sc_scale(logits, labx)
    return _tc_tail(sc_out, logits, labels_i32)


# speedup vs baseline: 1.0823x; 1.0823x over previous
import jax
import jax.numpy as jnp
from jax import lax
from jax.experimental import pallas as pl
from jax.experimental.pallas import tpu as pltpu

_S = 64.0
_M = 0.4

_RB = 8
_NBUF = 4


def kernel(logits, norms, labels):
    del norms
    b, c = logits.shape
    t_total = b // _RB
    labels2d = labels.astype(jnp.int32).reshape(b, 1)

    def body(labels_ref, logits_hbm, out_hbm, *scr):
        ins = scr[0:_NBUF]
        outs = scr[_NBUF:2 * _NBUF]
        lsems = scr[2 * _NBUF:3 * _NBUF]
        ssems = scr[3 * _NBUF:4 * _NBUF]

        def src_slice(t):
            return logits_hbm.at[pl.ds(t * _RB, _RB), :]

        def dst_slice(t):
            return out_hbm.at[pl.ds(t * _RB, _RB), :]

        for bb in range(_NBUF):
            pltpu.make_async_copy(src_slice(bb), ins[bb], lsems[bb]).start()

        cols = lax.broadcasted_iota(jnp.int32, (_RB, c), 1)

        def group(g, carry):
            for bb in range(_NBUF):
                t = g * _NBUF + bb
                ib, ob, ls, ss = ins[bb], outs[bb], lsems[bb], ssems[bb]
                pltpu.make_async_copy(src_slice(t), ib, ls).wait()

                @pl.when(t >= _NBUF)
                def _(ob=ob, ss=ss, t=t):
                    pltpu.make_async_copy(
                        ob, dst_slice(t - _NBUF), ss).wait()

                lab = labels_ref[pl.ds(t * _RB, _RB), :]
                x = ib[...]
                ob[...] = (x - jnp.where(cols == lab, _M, 0.0)) * _S

                pltpu.make_async_copy(ob, dst_slice(t), ss).start()

                @pl.when(t + _NBUF < t_total)
                def _(ib=ib, ls=ls, t=t):
                    pltpu.make_async_copy(src_slice(t + _NBUF), ib, ls).start()

            return carry

        lax.fori_loop(0, t_total // _NBUF, group, 0)
        for bb in range(_NBUF):
            pltpu.make_async_copy(
                outs[bb], dst_slice(t_total - _NBUF + bb), ssems[bb]).wait()

    return pl.pallas_call(
        body,
        in_specs=[
            pl.BlockSpec((b, 1), lambda: (0, 0)),
            pl.BlockSpec(memory_space=pltpu.HBM),
        ],
        out_specs=pl.BlockSpec(memory_space=pltpu.HBM),
        out_shape=jax.ShapeDtypeStruct((b, c), jnp.float32),
        scratch_shapes=(
            [pltpu.VMEM((_RB, c), jnp.float32)] * (2 * _NBUF)
            + [pltpu.SemaphoreType.DMA] * (2 * _NBUF)
        ),
    )(labels2d, logits)


# speedup vs baseline: 4.1122x; 3.7995x over previous
import jax
import jax.numpy as jnp
from jax.experimental import pallas as pl

_S = 64.0
_M = 0.4

_BLOCK_R = 2000


def _body(labels_ref, x_ref, o_ref):
    i = pl.program_id(0)
    br, b = x_ref.shape
    rows = i * _BLOCK_R + jax.lax.broadcasted_iota(jnp.int32, (br, b), 0)
    lab = labels_ref[...]
    x = x_ref[...]
    o_ref[...] = (x - jnp.where(rows == lab, _M, 0.0)) * _S


def kernel(logits, norms, labels):
    del norms
    b, c = logits.shape
    lt = logits.T
    labels_row = labels.astype(jnp.int32).reshape(1, b)
    out_t = pl.pallas_call(
        _body,
        grid=(c // _BLOCK_R,),
        in_specs=[
            pl.BlockSpec((1, b), lambda i: (0, 0)),
            pl.BlockSpec((_BLOCK_R, b), lambda i: (i, 0)),
        ],
        out_specs=pl.BlockSpec((_BLOCK_R, b), lambda i: (i, 0)),
        out_shape=jax.ShapeDtypeStruct((c, b), jnp.float32),
    )(labels_row, lt)
    return out_t.T
